# baseline (speedup 1.0000x reference)
import functools

import jax
import jax.numpy as jnp
from jax import lax
from jax.experimental import pallas as pl
from jax.experimental.pallas import tpu as pltpu
from jax.experimental.pallas import tpu_sc as plsc


def _make_sc_gather(V, D, N):
    info = plsc.get_sparse_core_info()
    NC, NS = info.num_cores, info.num_subcores
    NW = NC * NS
    assert N % NW == 0
    per_w = N // NW
    sizes = [128] * (per_w // 128)
    if per_w % 128:
        sizes.append(per_w % 128)
    mesh = plsc.VectorSubcoreMesh(core_axis_name="c", subcore_axis_name="s")

    @functools.partial(
        pl.kernel,
        out_type=jax.ShapeDtypeStruct((N, D), jnp.float32),
        mesh=mesh,
        compiler_params=pltpu.CompilerParams(use_tc_tiling_on_sc=False),
        scratch_types=[
            pltpu.VMEM((per_w,), jnp.int32),
            pltpu.VMEM((per_w, D), jnp.float32),
            pltpu.SemaphoreType.DMA,
        ],
    )
    def gather_k(table_hbm, idx_hbm, out_hbm, idx_v, rows_v, sem):
        wid = lax.axis_index("s") * NC + lax.axis_index("c")
        base = wid * per_w
        pltpu.sync_copy(idx_hbm.at[pl.ds(base, per_w)], idx_v)
        copies = []
        off = 0
        for sz in sizes:
            copies.append(
                pltpu.async_copy(
                    table_hbm.at[idx_v.at[pl.ds(off, sz)]],
                    rows_v.at[pl.ds(off, sz)],
                    sem,
                )
            )
            off += sz
        for cp in copies:
            cp.wait()
        pltpu.sync_copy(rows_v, out_hbm.at[pl.ds(base, per_w)])

    return gather_k


def _lstm_tc(x, Wk, Wr, b2):
    T, B, D = x.shape
    H4 = Wk.shape[1]
    H = H4 // 4

    def body(x_ref, wk_ref, wr_ref, b_ref, y_ref, hl_ref, h_s, c_s):
        t = pl.program_id(0)

        @pl.when(t == 0)
        def _():
            h_s[:] = jnp.zeros_like(h_s)
            c_s[:] = jnp.zeros_like(c_s)

        z = (
            jnp.dot(x_ref[0], wk_ref[:], preferred_element_type=jnp.float32)
            + jnp.dot(h_s[:], wr_ref[:], preferred_element_type=jnp.float32)
            + b_ref[:]
        )
        i = jax.nn.sigmoid(z[:, :H])
        f = jax.nn.sigmoid(z[:, H : 2 * H])
        g = jnp.tanh(z[:, 2 * H : 3 * H])
        o = jax.nn.sigmoid(z[:, 3 * H :])
        c_new = f * c_s[:] + i * g
        h_new = o * jnp.tanh(c_new)
        c_s[:] = c_new
        h_s[:] = h_new
        y_ref[:, :] = h_new

        @pl.when(t == T - 1)
        def _():
            hl_ref[:] = h_new

    y_flat, hl = pl.pallas_call(
        body,
        grid=(T,),
        in_specs=[
            pl.BlockSpec((1, B, D), lambda t: (t, 0, 0)),
            pl.BlockSpec((D, H4), lambda t: (0, 0)),
            pl.BlockSpec((H, H4), lambda t: (0, 0)),
            pl.BlockSpec((1, H4), lambda t: (0, 0)),
        ],
        out_specs=[
            pl.BlockSpec((B, H), lambda t: (0, t)),
            pl.BlockSpec((B, H), lambda t: (0, 0)),
        ],
        out_shape=[
            jax.ShapeDtypeStruct((B, T * H), jnp.float32),
            jax.ShapeDtypeStruct((B, H), jnp.float32),
        ],
        scratch_shapes=[
            pltpu.VMEM((B, H), jnp.float32),
            pltpu.VMEM((B, H), jnp.float32),
        ],
    )(x, Wk, Wr, b2)
    return y_flat.reshape(B, T, H), hl


def kernel(inputs, E, Wk, Wr, b):
    B, T = inputs.shape
    V, D = E.shape
    idx = jnp.transpose(inputs).reshape(-1).astype(jnp.int32)
    x_flat = _make_sc_gather(V, D, T * B)(E, idx)
    x = x_flat.reshape(T, B, D)
    y, h_last = _lstm_tc(x, Wk, Wr, b.reshape(1, -1))
    return (y, h_last)


# speedup vs baseline: 1.1829x; 1.1829x over previous
import functools

import jax
import jax.numpy as jnp
from jax import lax
from jax.experimental import pallas as pl
from jax.experimental.pallas import tpu as pltpu
from jax.experimental.pallas import tpu_sc as plsc


def _make_sc_gather(V, D, N):
    info = plsc.get_sparse_core_info()
    NC, NS, L = info.num_cores, info.num_subcores, info.num_lanes
    NW = NC * NS
    assert N % (2 * NW) == 0
    per_w = N // NW
    assert per_w % L == 0
    n_b = per_w // L
    mesh = plsc.VectorSubcoreMesh(core_axis_name="c", subcore_axis_name="s")

    @functools.partial(
        pl.kernel,
        out_type=jax.ShapeDtypeStruct((N // 2, 2 * D), jnp.float32),
        mesh=mesh,
        compiler_params=pltpu.CompilerParams(needs_layout_passes=False),
        scratch_types=[
            pltpu.VMEM((per_w,), jnp.int32),
            pltpu.VMEM((L * 8, D), jnp.float32),
            pltpu.VMEM((per_w // 2, 2 * D), jnp.float32),
            pltpu.SemaphoreType.DMA,
        ],
    )
    def gather_k(table_hbm, idx_hbm, out_hbm, idx_v, slabs_v, rows_v, sem):
        wid = lax.axis_index("s") * NC + lax.axis_index("c")
        base = pl.multiple_of(wid * per_w, per_w)
        pltpu.sync_copy(idx_hbm.at[pl.ds(base, per_w)], idx_v)
        lane = lax.iota(jnp.int32, L)

        def batch_body(b):
            v = idx_v[pl.ds(b * L, L)]
            copies = [
                pltpu.async_copy(
                    table_hbm.at[pl.ds(pl.multiple_of((v[j] >> 3) * 8, 8), 8)],
                    slabs_v.at[pl.ds(j * 8, 8)],
                    sem,
                )
                for j in range(L)
            ]
            for cp in copies:
                cp.wait()
            for j in range(L):
                r16 = jnp.full((L,), j * 8, jnp.int32) + (v[j] & 7)
                q16 = jnp.full((L,), b * (L // 2) + j // 2, jnp.int32)
                for c0 in range(0, D, L):
                    vals = plsc.load_gather(slabs_v, [r16, c0 + lane])
                    plsc.store_scatter(
                        rows_v, [q16, (j % 2) * D + c0 + lane], vals
                    )

        pl.loop(0, n_b)(batch_body)
        obase = pl.multiple_of(wid * (per_w // 2), per_w // 2)
        pltpu.sync_copy(rows_v, out_hbm.at[pl.ds(obase, per_w // 2)])

    return gather_k


def _lstm_tc(x2, Wk, Wr, b2):
    T, B2, D2 = x2.shape
    B = 2 * B2
    D = D2 // 2
    H4 = Wk.shape[1]
    H = H4 // 4

    def body(x_ref, wk_ref, wr_ref, b_ref, y_ref, hl_ref, h_s, c_s):
        t = pl.program_id(0)

        @pl.when(t == 0)
        def _():
            h_s[:] = jnp.zeros_like(h_s)
            c_s[:] = jnp.zeros_like(c_s)

        xt2 = x_ref[0]
        xt = jnp.concatenate([xt2[:, :D], xt2[:, D:]], axis=0)
        z = (
            jnp.dot(xt, wk_ref[:], preferred_element_type=jnp.float32)
            + jnp.dot(h_s[:], wr_ref[:], preferred_element_type=jnp.float32)
            + b_ref[:]
        )
        i = jax.nn.sigmoid(z[:, :H])
        f = jax.nn.sigmoid(z[:, H : 2 * H])
        g = jnp.tanh(z[:, 2 * H : 3 * H])
        o = jax.nn.sigmoid(z[:, 3 * H :])
        c_new = f * c_s[:] + i * g
        h_new = o * jnp.tanh(c_new)
        c_s[:] = c_new
        h_s[:] = h_new
        y_ref[:, :] = h_new

        @pl.when(t == T - 1)
        def _():
            hl_ref[:] = h_new

    y_flat, hl = pl.pallas_call(
        body,
        grid=(T,),
        in_specs=[
            pl.BlockSpec((1, B2, D2), lambda t: (t, 0, 0)),
            pl.BlockSpec((D, H4), lambda t: (0, 0)),
            pl.BlockSpec((H, H4), lambda t: (0, 0)),
            pl.BlockSpec((1, H4), lambda t: (0, 0)),
        ],
        out_specs=[
            pl.BlockSpec((B, H), lambda t: (0, t)),
            pl.BlockSpec((B, H), lambda t: (0, 0)),
        ],
        out_shape=[
            jax.ShapeDtypeStruct((B, T * H), jnp.float32),
            jax.ShapeDtypeStruct((B, H), jnp.float32),
        ],
        scratch_shapes=[
            pltpu.VMEM((B, H), jnp.float32),
            pltpu.VMEM((B, H), jnp.float32),
        ],
    )(x2, Wk, Wr, b2)
    return y_flat.reshape(B, T, H), hl


def kernel(inputs, E, Wk, Wr, b):
    B, T = inputs.shape
    V, D = E.shape
    idx = jnp.transpose(inputs).reshape(-1).astype(jnp.int32)
    idx_arr = idx.reshape(T, 2, B // 2).transpose(0, 2, 1).reshape(-1)
    x_packed = _make_sc_gather(V, D, T * B)(E, idx_arr)
    x2 = x_packed.reshape(T, B // 2, 2 * D)
    y, h_last = _lstm_tc(x2, Wk, Wr, b.reshape(1, -1))
    return (y, h_last)
